# baseline (speedup 1.0000x reference)
import jax
import jax.numpy as jnp
from jax import lax
from jax.experimental import pallas as pl
from jax.experimental.pallas import tpu as pltpu

R = 128
C = 32768
BC = 8192
NB = C // BC
G = 4
RG = R // G

_BIG = 2**30


def _body(x_ref, o_ref, m_ref, i_ref):
    p = pl.program_id(0)
    b = pl.program_id(1)

    @pl.when(p < G)
    def _pass0():
        x = x_ref[...]
        bm = jnp.max(x, axis=1, keepdims=True)
        col = lax.broadcasted_iota(jnp.int32, x.shape, 1) + b * BC
        bi = jnp.min(jnp.where(x == bm, col, _BIG), axis=1, keepdims=True)
        rows = pl.ds(p * RG, RG)

        @pl.when(b == 0)
        def _():
            m_ref[rows] = bm
            i_ref[rows] = bi

        @pl.when(b != 0)
        def _():
            better = bm > m_ref[rows]
            m_ref[rows] = jnp.where(better, bm, m_ref[rows])
            i_ref[rows] = jnp.where(better, bi, i_ref[rows])

    @pl.when(p >= 1)
    def _pass1():
        col = lax.broadcasted_iota(jnp.int32, o_ref.shape, 1) + b * BC
        o_ref[...] = (col == i_ref[pl.ds((p - 1) * RG, RG)]).astype(jnp.float32)


def kernel(input):
    return pl.pallas_call(
        _body,
        grid=(G + 1, NB),
        in_specs=[
            pl.BlockSpec(
                (RG, BC),
                lambda p, b: (jnp.minimum(p, G - 1),
                              jnp.where(p < G, b, NB - 1)),
            ),
        ],
        out_specs=pl.BlockSpec(
            (RG, BC),
            lambda p, b: (jnp.maximum(p, 1) - 1, jnp.where(p >= 1, b, 0)),
        ),
        out_shape=jax.ShapeDtypeStruct((R, C), jnp.float32),
        scratch_shapes=[
            pltpu.VMEM((R, 1), jnp.float32),
            pltpu.VMEM((R, 1), jnp.int32),
        ],
    )(input)
